# baseline (speedup 1.0000x reference)
import functools

import jax
import jax.numpy as jnp
from jax.experimental import pallas as pl
from jax.experimental.pallas import tpu as pltpu
import jax.experimental.pallas.tpu_sc as plsc

NG = 2
NV = 320
GV = NG * NV
VD = 128

_SC_CORES = 2
_SC_SUBCORES = 16
_SC_TILES = _SC_CORES * _SC_SUBCORES


def _tc_kernel(x_ref, wt_ref, b_ref, i0_ref, i1_ref, cp_ref, pp_ref,
               hist_ref, psum_ref, *, total_rows):
    step = pl.program_id(0)
    nsteps = pl.num_programs(0)

    x = x_ref[...]
    logits = jnp.dot(x, wt_ref[...],
                     preferred_element_type=jnp.float32) + b_ref[...]

    col = jax.lax.broadcasted_iota(jnp.int32, logits.shape, 1)
    g0 = col < NV
    neg = jnp.float32(-jnp.inf)
    m0 = jnp.where(g0, logits, neg)
    m1 = jnp.where(g0, neg, logits)
    mx0 = jnp.max(m0, axis=1, keepdims=True)
    mx1 = jnp.max(m1, axis=1, keepdims=True)
    big = jnp.int32(GV)
    idx0 = jnp.min(jnp.where(m0 == mx0, col, big), axis=1, keepdims=True)
    idx1 = jnp.min(jnp.where(m1 == mx1, col, big), axis=1, keepdims=True)
    i0_ref[...] = idx0
    i1_ref[...] = idx1
    oh0 = (col == idx0).astype(jnp.float32)
    oh1 = (col == idx1).astype(jnp.float32)

    e0 = jnp.exp(m0 - mx0)
    e1 = jnp.exp(m1 - mx1)
    p = (e0 / jnp.sum(e0, axis=1, keepdims=True)
         + e1 / jnp.sum(e1, axis=1, keepdims=True))

    hist_inc = jnp.sum(oh0 + oh1, axis=0, keepdims=True)
    psum_inc = jnp.sum(p, axis=0, keepdims=True)

    @pl.when(step == 0)
    def _():
        hist_ref[...] = hist_inc
        psum_ref[...] = psum_inc

    @pl.when(step != 0)
    def _():
        hist_ref[...] += hist_inc
        psum_ref[...] += psum_inc

    @pl.when(step == nsteps - 1)
    def _():
        inv = jnp.float32(1.0 / total_rows)
        grow = jax.lax.broadcasted_iota(jnp.int32, (1, GV), 1) < NV

        def pplx(pr):
            t = pr * jnp.log(pr + 1e-7)
            s0 = jnp.sum(jnp.where(grow, t, 0.0))
            s1 = jnp.sum(jnp.where(grow, 0.0, t))
            return jnp.exp(-s0) + jnp.exp(-s1)

        cp_ref[...] = jnp.broadcast_to(pplx(hist_ref[...] * inv), (1, 1))
        pp_ref[...] = jnp.broadcast_to(pplx(psum_ref[...] * inv), (1, 1))


def _sc_gather(i0_hbm, i1_hbm, cb_hbm, q_hbm, idx0_v, idx1_v, buf0, buf1,
               *, rows_per_tile):
    c = jax.lax.axis_index("c")
    s = jax.lax.axis_index("s")
    tid = c * _SC_SUBCORES + s
    base = tid * rows_per_tile
    rpt = rows_per_tile
    pltpu.sync_copy(i0_hbm.at[pl.ds(base, rpt)], idx0_v)
    pltpu.sync_copy(i1_hbm.at[pl.ds(base, rpt)], idx1_v)
    pltpu.sync_copy(cb_hbm.at[idx0_v], buf0)
    pltpu.sync_copy(cb_hbm.at[idx1_v], buf1)
    pltpu.sync_copy(buf0, q_hbm.at[pl.ds(base, rpt), pl.ds(0, VD)])
    pltpu.sync_copy(buf1, q_hbm.at[pl.ds(base, rpt), pl.ds(VD, VD)])


def kernel(x, codebook, W, b):
    bsz, tsz, fsz = x.shape
    xf = x.reshape(-1, fsz)
    rows = xf.shape[0]
    R = 512
    grid = rows // R
    wt = W.T
    cb = codebook[0]
    b2 = b.reshape(1, GV)

    i0, i1, cp, pp = pl.pallas_call(
        functools.partial(_tc_kernel, total_rows=rows),
        grid=(grid,),
        in_specs=[
            pl.BlockSpec((R, fsz), lambda i: (i, 0)),
            pl.BlockSpec((fsz, GV), lambda i: (0, 0)),
            pl.BlockSpec((1, GV), lambda i: (0, 0)),
        ],
        out_specs=[
            pl.BlockSpec((R, 1), lambda i: (i, 0)),
            pl.BlockSpec((R, 1), lambda i: (i, 0)),
            pl.BlockSpec((1, 1), lambda i: (0, 0)),
            pl.BlockSpec((1, 1), lambda i: (0, 0)),
        ],
        out_shape=[
            jax.ShapeDtypeStruct((rows, 1), jnp.int32),
            jax.ShapeDtypeStruct((rows, 1), jnp.int32),
            jax.ShapeDtypeStruct((1, 1), jnp.float32),
            jax.ShapeDtypeStruct((1, 1), jnp.float32),
        ],
        scratch_shapes=[
            pltpu.VMEM((1, GV), jnp.float32),
            pltpu.VMEM((1, GV), jnp.float32),
        ],
    )(xf, wt, b2)

    rpt = rows // _SC_TILES
    sc_fn = pl.kernel(
        functools.partial(_sc_gather, rows_per_tile=rpt),
        out_type=jax.ShapeDtypeStruct((rows, NG * VD), jnp.float32),
        mesh=plsc.VectorSubcoreMesh(
            core_axis_name="c", subcore_axis_name="s",
            num_cores=_SC_CORES, num_subcores=_SC_SUBCORES),
        scratch_types=[
            pltpu.VMEM((rpt,), jnp.int32),
            pltpu.VMEM((rpt,), jnp.int32),
            pltpu.VMEM((rpt, VD), jnp.float32),
            pltpu.VMEM((rpt, VD), jnp.float32),
        ],
    )
    q = sc_fn(i0.reshape(rows), i1.reshape(rows), cb)

    return (q.reshape(bsz, tsz, NG * VD), codebook.shape[1],
            cp[0, 0], pp[0, 0])
